# Initial kernel scaffold; baseline (speedup 1.0000x reference)
import jax
import jax.numpy as jnp
from jax.experimental import pallas as pl


def kernel(h_x, t_x, W1, a1s, a1d, b1, Wi, ais, aid, bi, Ws, Wd, abs_a, abd_a, bb, Wp1, Wp2, bp, Wq, Wk, Wv, Wo, h_edge_index, t_edge_index, b_edge_index):
    raise NotImplementedError("write your pallas kernel here")


# speedup vs baseline: 17.9293x; 17.9293x over previous
import functools
import math

import jax
import jax.numpy as jnp
from jax import lax
from jax.experimental import pallas as pl
from jax.experimental.pallas import tpu as pltpu
from jax.experimental.pallas import tpu_sc as plsc

N_GRAPHS = 1024
N_PER = 50
N = N_GRAPHS * N_PER
E = 204800
EB = 524288
D_IN = 128
HEADS1 = 2
HO1 = 64
HI = 2
OI = 32
DREP = 2 * HI * OI
K_POOL = math.ceil(N_PER * 0.6)
DIMS = math.ceil(100 * 0.6)
NH_ATT = 8
DH = DREP // NH_ATT

_NC = 2
_NS = 16
_TILES = _NC * _NS
_GPT = N_GRAPHS // _TILES
_ROWS = _GPT * N_PER
_CNT = _ROWS * N_PER
_CHUNK = 8192


def _counts_body(hs, hd, ts, td, bs, bd, ch, ct, cb, cnt_v, s_v, d_v):
    wid = lax.axis_index("s") * _NC + lax.axis_index("c")
    row_lo = wid * _ROWS
    ones = jnp.full((16,), 1.0, jnp.float32)
    zeros = jnp.zeros((16,), jnp.float32)

    for (src_h, dst_h, out_h, ne) in ((hs, hd, ch, E), (ts, td, ct, E),
                                      (bs, bd, cb, EB)):
        def zbody(i, _):
            cnt_v[pl.ds(i * 16, 16)] = zeros
            return 0
        lax.fori_loop(0, _CNT // 16, zbody, 0)

        def chunk_body(c, _):
            pltpu.sync_copy(src_h.at[pl.ds(c * _CHUNK, _CHUNK)], s_v)
            pltpu.sync_copy(dst_h.at[pl.ds(c * _CHUNK, _CHUNK)], d_v)

            def vbody(i, _):
                s = s_v[pl.ds(i * 16, 16)]
                d = d_v[pl.ds(i * 16, 16)]
                q = lax.div(s, 50)
                srcl = s - q * 50
                r = d - row_lo
                msk = (r >= 0) & (r < _ROWS)
                idx = r * N_PER + srcl
                plsc.addupdate_scatter(cnt_v, [idx], ones, mask=msk)
                return 0
            lax.fori_loop(0, _CHUNK // 16, vbody, 0)
            return 0
        lax.fori_loop(0, ne // _CHUNK, chunk_body, 0)
        pltpu.sync_copy(cnt_v, out_h.at[pl.ds(wid * _CNT, _CNT)])


def _build_counts(hs, hd, ts, td, bs, bd):
    mesh = plsc.VectorSubcoreMesh(core_axis_name="c", subcore_axis_name="s")
    out = jax.ShapeDtypeStruct((N * N_PER,), jnp.float32)
    fn = pl.kernel(
        _counts_body,
        out_type=(out, out, out),
        mesh=mesh,
        scratch_types=[
            pltpu.VMEM((_CNT,), jnp.float32),
            pltpu.VMEM((_CHUNK,), jnp.int32),
            pltpu.VMEM((_CHUNK,), jnp.int32),
        ],
        compiler_params=pltpu.CompilerParams(needs_layout_passes=False),
    )
    return fn(hs, hd, ts, td, bs, bd)


GB = 8
GRID = N_GRAPHS // GB

_BD = (((2,), (1,)), ((0,), (0,)))
_BT = (((1,), (1,)), ((0,), (0,)))
_BK = (((2,), (2,)), ((0,), (0,)))
_PW = (((2,), (0,)), ((), ()))


_dot = functools.partial(lax.dot_general,
                         precision=lax.Precision.HIGHEST,
                         preferred_element_type=jnp.float32)
_dot_mirror = functools.partial(lax.dot_general,
                                precision=lax.Precision.DEFAULT,
                                preferred_element_type=jnp.float32)


def _leaky(x):
    return jnp.where(x >= 0, x, 0.2 * x)


def _gat_weights(e, c, axis):
    mask = c > 0.0
    m = jnp.max(jnp.where(mask, e, -1e30), axis=axis, keepdims=True)
    m = jnp.where(m > -1e29, m, 0.0)
    ex = jnp.where(mask, c * jnp.exp(e - m), 0.0)
    den = jnp.sum(ex, axis=axis, keepdims=True)
    return ex / (den + 1e-16)


def _gat_dense(xs, xd, c3, a_s, a_d, bias, heads, oc, self_loops, transposed):
    if self_loops:
        i50 = lax.broadcasted_iota(jnp.int32, (1, N_PER, N_PER), 1)
        j50 = lax.broadcasted_iota(jnp.int32, (1, N_PER, N_PER), 2)
        c3 = c3 + (i50 == j50).astype(jnp.float32)
    outs = []
    for h in range(heads):
        xs_h = xs[:, :, h * oc:(h + 1) * oc]
        xd_h = xd[:, :, h * oc:(h + 1) * oc]
        as_h = jnp.sum(xs_h * a_s[h].reshape(1, 1, oc), axis=2)
        ad_h = jnp.sum(xd_h * a_d[h].reshape(1, 1, oc), axis=2)
        if transposed:
            e = _leaky(as_h[:, :, None] + ad_h[:, None, :])
            w = _gat_weights(e, c3, axis=1)
            out_h = _dot(w, xs_h, _BT)
        else:
            e = _leaky(ad_h[:, :, None] + as_h[:, None, :])
            w = _gat_weights(e, c3, axis=2)
            out_h = _dot(w, xs_h, _BD)
        outs.append(out_h)
    return jnp.concatenate(outs, axis=2) + bias.reshape(1, 1, heads * oc)


def _proj(x, w):
    return _dot_mirror(x, w, _PW)


def _mega_body(hx, tx, ch, ct, cb,
               w1, a1s, a1d, b1, wi, ais, aid, bi,
               ws, wd, absa, abda, bb, wp1, wp2, bp,
               wq, wk, wv, wo,
               h_rep_o, t_rep_o, h_att_o, t_att_o):
    hx = hx[...]
    tx = tx[...]
    ch3 = ch[...]
    ct3 = ct[...]
    cb3 = cb[...]
    w1_ = w1[...]
    wi_ = wi[...]
    ws_ = ws[...]
    wd_ = wd[...]

    hxp = _proj(hx, w1_)
    txp = _proj(tx, w1_)
    h1 = _gat_dense(hxp, hxp, ch3, a1s[...], a1d[...], b1[...][0],
                    HEADS1, HO1, True, False)
    t1 = _gat_dense(txp, txp, ct3, a1s[...], a1d[...], b1[...][0],
                    HEADS1, HO1, True, False)
    h_act = jnp.where(h1 > 0, h1, jnp.exp(jnp.minimum(h1, 0.0)) - 1.0)
    t_act = jnp.where(t1 > 0, t1, jnp.exp(jnp.minimum(t1, 0.0)) - 1.0)

    hip = _proj(h_act, wi_)
    tip = _proj(t_act, wi_)
    h_intra = _gat_dense(hip, hip, ch3, ais[...], aid[...], bi[...][0],
                         HI, OI, True, False)
    t_intra = _gat_dense(tip, tip, ct3, ais[...], aid[...], bi[...][0],
                         HI, OI, True, False)

    h_s = _proj(h_act, ws_)
    t_d = _proj(t_act, wd_)
    t_inter = _gat_dense(h_s, t_d, cb3, absa[...], abda[...], bb[...][0],
                         HI, OI, False, False)
    t_s = _proj(t_act, ws_)
    h_d = _proj(h_act, wd_)
    h_inter = _gat_dense(t_s, h_d, cb3, absa[...], abda[...], bb[...][0],
                         HI, OI, False, True)

    h_rep = jnp.concatenate([h_intra, h_inter], axis=2)
    t_rep = jnp.concatenate([t_intra, t_inter], axis=2)
    h_rep_o[...] = h_rep
    t_rep_o[...] = t_rep

    wp1_ = wp1[...]
    wp2_ = wp2[...]
    bps = bp[...][0, 0]
    wq_ = wq[...]
    wk_ = wk[...]
    wv_ = wv[...]
    wo_ = wo[...]

    for rep, c3, out_ref in ((h_rep, ch3, h_att_o), (t_rep, ct3, t_att_o)):
        agg = _dot(c3, rep, _BD)
        score = (_dot_mirror(rep, wp1_, _PW)
                 + _dot_mirror(agg, wp2_, _PW) + bps)[:, :, 0]
        s_i = score[:, :, None]
        s_j = score[:, None, :]
        ii = lax.broadcasted_iota(jnp.int32, (1, N_PER, N_PER), 1)
        jj = lax.broadcasted_iota(jnp.int32, (1, N_PER, N_PER), 2)
        beats = (s_j > s_i) | ((s_j == s_i) & (jj < ii))
        rank = jnp.sum(beats.astype(jnp.int32), axis=2)
        rank = jnp.where(rank < K_POOL, rank, 63)
        kio = lax.broadcasted_iota(jnp.int32, (1, DIMS, N_PER), 1)
        perm = (kio == rank[:, None, :]).astype(jnp.float32)
        xg = _dot(perm, rep, _BD)
        vals = jnp.sum(perm * score[:, None, :], axis=2)
        xd = xg * jnp.tanh(vals)[:, :, None]

        q = _proj(xd, wq_)
        k = _proj(xd, wk_)
        v = _proj(xd, wv_)
        heads_out = []
        for h in range(NH_ATT):
            sl = slice(h * DH, (h + 1) * DH)
            qh = q[:, :, sl]
            kh = k[:, :, sl]
            vh = v[:, :, sl]
            sc = _dot_mirror(qh, kh, _BK) * (1.0 / 4.0)
            sc = sc - jnp.max(sc, axis=2, keepdims=True)
            p = jnp.exp(sc)
            p = p / jnp.sum(p, axis=2, keepdims=True)
            heads_out.append(_dot_mirror(p, vh, _BD))
        o = jnp.concatenate(heads_out, axis=2)
        out_ref[...] = _proj(o, wo_)


def _full(shape):
    nd = len(shape)
    return pl.BlockSpec(shape, lambda i: (0,) * nd)


def _dense_specs_in():
    return [
        pl.BlockSpec((GB, N_PER, D_IN), lambda i: (i, 0, 0)),
        pl.BlockSpec((GB, N_PER, D_IN), lambda i: (i, 0, 0)),
        pl.BlockSpec((GB, N_PER, N_PER), lambda i: (i, 0, 0)),
        pl.BlockSpec((GB, N_PER, N_PER), lambda i: (i, 0, 0)),
        pl.BlockSpec((GB, N_PER, N_PER), lambda i: (i, 0, 0)),
        _full((D_IN, HEADS1 * HO1)),
        _full((HEADS1, HO1)), _full((HEADS1, HO1)), _full((1, HEADS1 * HO1)),
        _full((HEADS1 * HO1, HI * OI)),
        _full((HI, OI)), _full((HI, OI)), _full((1, HI * OI)),
        _full((HEADS1 * HO1, HI * OI)), _full((HEADS1 * HO1, HI * OI)),
        _full((HI, OI)), _full((HI, OI)), _full((1, HI * OI)),
        _full((DREP, 1)), _full((DREP, 1)), _full((1, 1)),
        _full((DREP, DREP)), _full((DREP, DREP)),
        _full((DREP, DREP)), _full((DREP, DREP)),
    ]


def _dense_specs_out():
    return [
        pl.BlockSpec((GB, N_PER, DREP), lambda i: (i, 0, 0)),
        pl.BlockSpec((GB, N_PER, DREP), lambda i: (i, 0, 0)),
        pl.BlockSpec((GB, DIMS, DREP), lambda i: (i, 0, 0)),
        pl.BlockSpec((GB, DIMS, DREP), lambda i: (i, 0, 0)),
    ]


def _dense_out_shape():
    f32 = jnp.float32
    return [
        jax.ShapeDtypeStruct((N_GRAPHS, N_PER, DREP), f32),
        jax.ShapeDtypeStruct((N_GRAPHS, N_PER, DREP), f32),
        jax.ShapeDtypeStruct((N_GRAPHS, DIMS, DREP), f32),
        jax.ShapeDtypeStruct((N_GRAPHS, DIMS, DREP), f32),
    ]


@jax.jit
def _dense_stage(hx3, tx3, ch3, ct3, cb3, w1, a1s, a1d, b1, wi, ais, aid, bi,
                 ws, wd, absa, abda, bb, wp1, wp2, bp, wq, wk, wv, wo):
    return pl.pallas_call(
        _mega_body,
        grid=(GRID,),
        in_specs=_dense_specs_in(),
        out_specs=_dense_specs_out(),
        out_shape=_dense_out_shape(),
    )(hx3, tx3, ch3, ct3, cb3, w1, a1s, a1d, b1, wi, ais, aid, bi,
      ws, wd, absa, abda, bb, wp1, wp2, bp, wq, wk, wv, wo)


def kernel(h_x, t_x, W1, a1s, a1d, b1, Wi, ais, aid, bi, Ws, Wd, abs_a, abd_a,
           bb, Wp1, Wp2, bp, Wq, Wk, Wv, Wo,
           h_edge_index, t_edge_index, b_edge_index):
    i32 = jnp.int32
    hs = h_edge_index[0].astype(i32)
    hd = h_edge_index[1].astype(i32)
    ts = t_edge_index[0].astype(i32)
    td = t_edge_index[1].astype(i32)
    bs = b_edge_index[0].astype(i32)
    bd = b_edge_index[1].astype(i32)

    ch, ct, cb = _build_counts(hs, hd, ts, td, bs, bd)
    ch3 = ch.reshape(N_GRAPHS, N_PER, N_PER)
    ct3 = ct.reshape(N_GRAPHS, N_PER, N_PER)
    cb3 = cb.reshape(N_GRAPHS, N_PER, N_PER)

    f32 = jnp.float32
    hx3 = h_x.astype(f32).reshape(N_GRAPHS, N_PER, D_IN)
    tx3 = t_x.astype(f32).reshape(N_GRAPHS, N_PER, D_IN)

    h_rep3, t_rep3, h_att, t_att = _dense_stage(
        hx3, tx3, ch3, ct3, cb3,
        W1.astype(f32), a1s.astype(f32), a1d.astype(f32),
        b1.astype(f32).reshape(1, HEADS1 * HO1),
        Wi.astype(f32), ais.astype(f32), aid.astype(f32),
        bi.astype(f32).reshape(1, HI * OI),
        Ws.astype(f32), Wd.astype(f32), abs_a.astype(f32), abd_a.astype(f32),
        bb.astype(f32).reshape(1, HI * OI),
        Wp1.astype(f32), Wp2.astype(f32),
        bp.astype(f32).reshape(1, 1),
        Wq.astype(f32), Wk.astype(f32), Wv.astype(f32), Wo.astype(f32))

    return (h_rep3.reshape(N, DREP), t_rep3.reshape(N, DREP), h_att, t_att)


# speedup vs baseline: 43.1282x; 2.4055x over previous
import functools
import math

import jax
import jax.numpy as jnp
from jax import lax
from jax.experimental import pallas as pl
from jax.experimental.pallas import tpu as pltpu
from jax.experimental.pallas import tpu_sc as plsc

N_GRAPHS = 1024
N_PER = 50
N = N_GRAPHS * N_PER
E = 204800
EB = 524288
D_IN = 128
HEADS1 = 2
HO1 = 64
HI = 2
OI = 32
DREP = 2 * HI * OI
K_POOL = math.ceil(N_PER * 0.6)
DIMS = math.ceil(100 * 0.6)
NH_ATT = 8
DH = DREP // NH_ATT

_NC = 2
_NS = 16
_TILES = _NC * _NS
_GPT = N_GRAPHS // _TILES
_ROWS = _GPT * N_PER
_CNT = _ROWS * N_PER
_CHUNK = 16384


def _counts_body(hs, hd, ts, td, bs, bd, ch, ct, cb, cnt_v, s_v, d_v):
    wid = lax.axis_index("s") * _NC + lax.axis_index("c")
    row_lo = wid * _ROWS
    ones = jnp.full((16,), 1.0, jnp.float32)
    zeros = jnp.zeros((16,), jnp.float32)

    for (src_h, dst_h, out_h, ne) in ((hs, hd, ch, E), (ts, td, ct, E),
                                      (bs, bd, cb, EB)):
        def zbody(i, _):
            for u in range(8):
                cnt_v[pl.ds(i * 128 + u * 16, 16)] = zeros
            return 0
        lax.fori_loop(0, _CNT // 128, zbody, 0)

        def chunk_body(c, _):
            pltpu.sync_copy(src_h.at[pl.ds(c * _CHUNK, _CHUNK)], s_v)
            pltpu.sync_copy(dst_h.at[pl.ds(c * _CHUNK, _CHUNK)], d_v)

            def vbody(i, _):
                for u in range(4):
                    s = s_v[pl.ds(i * 64 + u * 16, 16)]
                    d = d_v[pl.ds(i * 64 + u * 16, 16)]
                    q = lax.div(s, 50)
                    srcl = s - q * 50
                    r = d - row_lo
                    msk = (r >= 0) & (r < _ROWS)
                    idx = r * N_PER + srcl
                    plsc.addupdate_scatter(cnt_v, [idx], ones, mask=msk)
                return 0
            lax.fori_loop(0, _CHUNK // 64, vbody, 0)
            return 0
        lax.fori_loop(0, ne // _CHUNK, chunk_body, 0)
        pltpu.sync_copy(cnt_v, out_h.at[pl.ds(wid * _CNT, _CNT)])


def _build_counts(hs, hd, ts, td, bs, bd):
    mesh = plsc.VectorSubcoreMesh(core_axis_name="c", subcore_axis_name="s")
    out = jax.ShapeDtypeStruct((N * N_PER,), jnp.float32)
    fn = pl.kernel(
        _counts_body,
        out_type=(out, out, out),
        mesh=mesh,
        scratch_types=[
            pltpu.VMEM((_CNT,), jnp.float32),
            pltpu.VMEM((_CHUNK,), jnp.int32),
            pltpu.VMEM((_CHUNK,), jnp.int32),
        ],
        compiler_params=pltpu.CompilerParams(needs_layout_passes=False),
    )
    return fn(hs, hd, ts, td, bs, bd)


GB = 8
GRID = N_GRAPHS // GB

_BD = (((2,), (1,)), ((0,), (0,)))
_TR = (((1,), (0,)), ((), ()))
_BT = (((1,), (1,)), ((0,), (0,)))
_BK = (((2,), (2,)), ((0,), (0,)))
_PW = (((2,), (0,)), ((), ()))


_dot = functools.partial(lax.dot_general,
                         precision=lax.Precision.HIGHEST,
                         preferred_element_type=jnp.float32)
_dot_mirror = functools.partial(lax.dot_general,
                                precision=lax.Precision.DEFAULT,
                                preferred_element_type=jnp.float32)


def _leaky(x):
    return jnp.where(x >= 0, x, 0.2 * x)


def _gat_weights(e, c, axis):
    mask = c > 0.0
    m = jnp.max(jnp.where(mask, e, -1e30), axis=axis, keepdims=True)
    m = jnp.where(m > -1e29, m, 0.0)
    ex = jnp.where(mask, c * jnp.exp(e - m), 0.0)
    if axis == 2:
        ones_l = jnp.ones((N_PER, 1), jnp.float32)
        den = _dot(ex, ones_l, _PW)
    else:
        den = jnp.sum(ex, axis=1, keepdims=True)
    return ex / (den + 1e-16)


def _gat_dense(xs, xd, c3, a_s, a_d, bias, heads, oc, self_loops, transposed):
    if self_loops:
        i50 = lax.broadcasted_iota(jnp.int32, (1, N_PER, N_PER), 1)
        j50 = lax.broadcasted_iota(jnp.int32, (1, N_PER, N_PER), 2)
        c3 = c3 + (i50 == j50).astype(jnp.float32)
    eye = (lax.broadcasted_iota(jnp.int32, (N_PER, N_PER), 0)
           == lax.broadcasted_iota(jnp.int32, (N_PER, N_PER), 1)
           ).astype(jnp.float32)
    outs = []
    for h in range(heads):
        xs_h = xs[:, :, h * oc:(h + 1) * oc]
        xd_h = xd[:, :, h * oc:(h + 1) * oc]
        as_s = jnp.sum(xs_h * a_s[h].reshape(1, 1, oc), axis=2,
                       keepdims=True)
        ad_s = jnp.sum(xd_h * a_d[h].reshape(1, 1, oc), axis=2,
                       keepdims=True)
        if transposed:
            ad_l = _dot(ad_s, eye, _TR)
            e = _leaky(as_s + ad_l)
            w = _gat_weights(e, c3, axis=1)
            out_h = _dot(w, xs_h, _BT)
        else:
            as_l = _dot(as_s, eye, _TR)
            e = _leaky(ad_s + as_l)
            w = _gat_weights(e, c3, axis=2)
            out_h = _dot(w, xs_h, _BD)
        outs.append(out_h)
    return jnp.concatenate(outs, axis=2) + bias.reshape(1, 1, heads * oc)


def _proj(x, w):
    return _dot_mirror(x, w, _PW)


def _mega_body(hx, tx, ch, ct, cb,
               w1, a1s, a1d, b1, wi, ais, aid, bi,
               ws, wd, absa, abda, bb, wp1, wp2, bp,
               wq, wk, wv, wo,
               h_rep_o, t_rep_o, h_att_o, t_att_o):
    hx = hx[...]
    tx = tx[...]
    ch3 = ch[...]
    ct3 = ct[...]
    cb3 = cb[...]
    w1_ = w1[...]
    wi_ = wi[...]
    ws_ = ws[...]
    wd_ = wd[...]

    hxp = _proj(hx, w1_)
    txp = _proj(tx, w1_)
    h1 = _gat_dense(hxp, hxp, ch3, a1s[...], a1d[...], b1[...][0],
                    HEADS1, HO1, True, False)
    t1 = _gat_dense(txp, txp, ct3, a1s[...], a1d[...], b1[...][0],
                    HEADS1, HO1, True, False)
    h_act = jnp.where(h1 > 0, h1, jnp.exp(jnp.minimum(h1, 0.0)) - 1.0)
    t_act = jnp.where(t1 > 0, t1, jnp.exp(jnp.minimum(t1, 0.0)) - 1.0)

    hip = _proj(h_act, wi_)
    tip = _proj(t_act, wi_)
    h_intra = _gat_dense(hip, hip, ch3, ais[...], aid[...], bi[...][0],
                         HI, OI, True, False)
    t_intra = _gat_dense(tip, tip, ct3, ais[...], aid[...], bi[...][0],
                         HI, OI, True, False)

    h_s = _proj(h_act, ws_)
    t_d = _proj(t_act, wd_)
    t_inter = _gat_dense(h_s, t_d, cb3, absa[...], abda[...], bb[...][0],
                         HI, OI, False, False)
    t_s = _proj(t_act, ws_)
    h_d = _proj(h_act, wd_)
    h_inter = _gat_dense(t_s, h_d, cb3, absa[...], abda[...], bb[...][0],
                         HI, OI, False, True)

    h_rep = jnp.concatenate([h_intra, h_inter], axis=2)
    t_rep = jnp.concatenate([t_intra, t_inter], axis=2)
    h_rep_o[...] = h_rep
    t_rep_o[...] = t_rep

    wp1_ = wp1[...]
    wp2_ = wp2[...]
    bps = bp[...][0, 0]
    wq_ = wq[...]
    wk_ = wk[...]
    wv_ = wv[...]
    wo_ = wo[...]

    eye = (lax.broadcasted_iota(jnp.int32, (N_PER, N_PER), 0)
           == lax.broadcasted_iota(jnp.int32, (N_PER, N_PER), 1)
           ).astype(jnp.float32)
    ones_l = jnp.ones((N_PER, 1), jnp.float32)
    for rep, c3, out_ref in ((h_rep, ch3, h_att_o), (t_rep, ct3, t_att_o)):
        agg = _dot(c3, rep, _BD)
        score_s = (_dot_mirror(rep, wp1_, _PW)
                   + _dot_mirror(agg, wp2_, _PW) + bps)
        score_l = _dot(score_s, eye, _TR)
        ii = lax.broadcasted_iota(jnp.int32, (1, N_PER, N_PER), 1)
        jj = lax.broadcasted_iota(jnp.int32, (1, N_PER, N_PER), 2)
        beats = (score_l > score_s) | ((score_l == score_s) & (jj < ii))
        rank_s = _dot(beats.astype(jnp.float32), ones_l, _PW)
        rank_s = jnp.where(rank_s < K_POOL, rank_s, 63.0)
        rank_l = _dot(rank_s, eye, _TR).astype(jnp.int32)
        kio = lax.broadcasted_iota(jnp.int32, (1, DIMS, N_PER), 1)
        perm = (kio == rank_l).astype(jnp.float32)
        xg = _dot(perm, rep, _BD)
        vals = _dot(perm, score_s, _BD)
        xd = xg * jnp.tanh(vals)

        q = _proj(xd, wq_)
        k = _proj(xd, wk_)
        v = _proj(xd, wv_)
        heads_out = []
        for h in range(NH_ATT):
            sl = slice(h * DH, (h + 1) * DH)
            qh = q[:, :, sl]
            kh = k[:, :, sl]
            vh = v[:, :, sl]
            sc = _dot_mirror(qh, kh, _BK) * (1.0 / 4.0)
            sc = sc - jnp.max(sc, axis=2, keepdims=True)
            p = jnp.exp(sc)
            p = p / jnp.sum(p, axis=2, keepdims=True)
            heads_out.append(_dot_mirror(p, vh, _BD))
        o = jnp.concatenate(heads_out, axis=2)
        out_ref[...] = _proj(o, wo_)


def _full(shape):
    nd = len(shape)
    return pl.BlockSpec(shape, lambda i: (0,) * nd)


def _dense_specs_in():
    return [
        pl.BlockSpec((GB, N_PER, D_IN), lambda i: (i, 0, 0)),
        pl.BlockSpec((GB, N_PER, D_IN), lambda i: (i, 0, 0)),
        pl.BlockSpec((GB, N_PER, N_PER), lambda i: (i, 0, 0)),
        pl.BlockSpec((GB, N_PER, N_PER), lambda i: (i, 0, 0)),
        pl.BlockSpec((GB, N_PER, N_PER), lambda i: (i, 0, 0)),
        _full((D_IN, HEADS1 * HO1)),
        _full((HEADS1, HO1)), _full((HEADS1, HO1)), _full((1, HEADS1 * HO1)),
        _full((HEADS1 * HO1, HI * OI)),
        _full((HI, OI)), _full((HI, OI)), _full((1, HI * OI)),
        _full((HEADS1 * HO1, HI * OI)), _full((HEADS1 * HO1, HI * OI)),
        _full((HI, OI)), _full((HI, OI)), _full((1, HI * OI)),
        _full((DREP, 1)), _full((DREP, 1)), _full((1, 1)),
        _full((DREP, DREP)), _full((DREP, DREP)),
        _full((DREP, DREP)), _full((DREP, DREP)),
    ]


def _dense_specs_out():
    return [
        pl.BlockSpec((GB, N_PER, DREP), lambda i: (i, 0, 0)),
        pl.BlockSpec((GB, N_PER, DREP), lambda i: (i, 0, 0)),
        pl.BlockSpec((GB, DIMS, DREP), lambda i: (i, 0, 0)),
        pl.BlockSpec((GB, DIMS, DREP), lambda i: (i, 0, 0)),
    ]


def _dense_out_shape():
    f32 = jnp.float32
    return [
        jax.ShapeDtypeStruct((N_GRAPHS, N_PER, DREP), f32),
        jax.ShapeDtypeStruct((N_GRAPHS, N_PER, DREP), f32),
        jax.ShapeDtypeStruct((N_GRAPHS, DIMS, DREP), f32),
        jax.ShapeDtypeStruct((N_GRAPHS, DIMS, DREP), f32),
    ]


@jax.jit
def _dense_stage(hx3, tx3, ch3, ct3, cb3, w1, a1s, a1d, b1, wi, ais, aid, bi,
                 ws, wd, absa, abda, bb, wp1, wp2, bp, wq, wk, wv, wo):
    return pl.pallas_call(
        _mega_body,
        grid=(GRID,),
        in_specs=_dense_specs_in(),
        out_specs=_dense_specs_out(),
        out_shape=_dense_out_shape(),
    )(hx3, tx3, ch3, ct3, cb3, w1, a1s, a1d, b1, wi, ais, aid, bi,
      ws, wd, absa, abda, bb, wp1, wp2, bp, wq, wk, wv, wo)


def kernel(h_x, t_x, W1, a1s, a1d, b1, Wi, ais, aid, bi, Ws, Wd, abs_a, abd_a,
           bb, Wp1, Wp2, bp, Wq, Wk, Wv, Wo,
           h_edge_index, t_edge_index, b_edge_index):
    i32 = jnp.int32
    hs = h_edge_index[0].astype(i32)
    hd = h_edge_index[1].astype(i32)
    ts = t_edge_index[0].astype(i32)
    td = t_edge_index[1].astype(i32)
    bs = b_edge_index[0].astype(i32)
    bd = b_edge_index[1].astype(i32)

    ch, ct, cb = _build_counts(hs, hd, ts, td, bs, bd)
    ch3 = ch.reshape(N_GRAPHS, N_PER, N_PER)
    ct3 = ct.reshape(N_GRAPHS, N_PER, N_PER)
    cb3 = cb.reshape(N_GRAPHS, N_PER, N_PER)

    f32 = jnp.float32
    hx3 = h_x.astype(f32).reshape(N_GRAPHS, N_PER, D_IN)
    tx3 = t_x.astype(f32).reshape(N_GRAPHS, N_PER, D_IN)

    h_rep3, t_rep3, h_att, t_att = _dense_stage(
        hx3, tx3, ch3, ct3, cb3,
        W1.astype(f32), a1s.astype(f32), a1d.astype(f32),
        b1.astype(f32).reshape(1, HEADS1 * HO1),
        Wi.astype(f32), ais.astype(f32), aid.astype(f32),
        bi.astype(f32).reshape(1, HI * OI),
        Ws.astype(f32), Wd.astype(f32), abs_a.astype(f32), abd_a.astype(f32),
        bb.astype(f32).reshape(1, HI * OI),
        Wp1.astype(f32), Wp2.astype(f32),
        bp.astype(f32).reshape(1, 1),
        Wq.astype(f32), Wk.astype(f32), Wv.astype(f32), Wo.astype(f32))

    return (h_rep3.reshape(N, DREP), t_rep3.reshape(N, DREP), h_att, t_att)
